# Initial kernel scaffold; baseline (speedup 1.0000x reference)
import functools

import jax
import jax.numpy as jnp
from jax import lax
from jax.experimental import pallas as pl
from jax.experimental.pallas import tpu as pltpu
from jax.experimental.pallas import tpu_sc as plsc

_NC = 2
_NS = 16
_LANES = 16


def _densify_body(n, nnz, blk_rows, rows_hbm, cols_hbm, vals_hbm, out_hbm,
                  rows_v, cols_v, vals_v, offs_v, upds_v, zbuf_v, blk_sh):
    cid = lax.axis_index("c")
    sid = lax.axis_index("s")
    ept = nnz // _NS
    blk_cells = blk_rows * n
    cells_per_tec = blk_cells // _NS
    n_blocks = n // blk_rows
    blocks_per_core = n_blocks // _NC
    n_vregs = ept // _LANES
    scat_w = offs_v.shape[1]
    vregs_per_scat = scat_w // _LANES

    base_e = sid * ept
    pltpu.sync_copy(rows_hbm.at[pl.ds(base_e, ept)], rows_v)
    pltpu.sync_copy(cols_hbm.at[pl.ds(base_e, ept)], cols_v)
    pltpu.sync_copy(vals_hbm.at[pl.ds(base_e, ept)], vals_v)

    def _z(i, _):
        zbuf_v[pl.ds(i * _LANES, _LANES)] = jnp.zeros((_LANES,), jnp.float32)
        return ()
    lax.fori_loop(0, cells_per_tec // _LANES, _z, ())

    def _block(bi, _):
        blk = cid * blocks_per_core + bi
        base_r = blk * blk_rows

        pltpu.sync_copy(zbuf_v, blk_sh.at[pl.ds(sid * cells_per_tec,
                                                cells_per_tec)])
        plsc.subcore_barrier()

        def _prep(v, _):
            r = rows_v[pl.ds(v * _LANES, _LANES)]
            c = cols_v[pl.ds(v * _LANES, _LANES)]
            w = vals_v[pl.ds(v * _LANES, _LANES)]
            rel = r - base_r
            inblk = (rel >= 0) & (rel < blk_rows)
            off = jnp.where(inblk, rel * n + c, 0)
            val = jnp.where(inblk, w, jnp.zeros((_LANES,), jnp.float32))
            row = v // vregs_per_scat
            colo = (v % vregs_per_scat) * _LANES
            offs_v[row, pl.ds(colo, _LANES)] = off
            upds_v[row, pl.ds(colo, _LANES)] = val
            return ()
        lax.fori_loop(0, n_vregs, _prep, ())

        def _scat(j, _):
            pltpu.sync_copy(upds_v.at[j], blk_sh.at[offs_v.at[j]], add=True)
            return ()
        lax.fori_loop(0, ept // scat_w, _scat, ())
        plsc.subcore_barrier()

        pltpu.sync_copy(
            blk_sh.at[pl.ds(sid * cells_per_tec, cells_per_tec)],
            out_hbm.at[pl.ds(base_r * n + sid * cells_per_tec,
                             cells_per_tec)])
        plsc.subcore_barrier()
        return ()

    lax.fori_loop(0, blocks_per_core, _block, ())


def _densify(rows, cols, vals, n):
    nnz = rows.shape[0]
    blk_rows = 256
    blk_cells = blk_rows * n
    ept = nnz // _NS
    scat_w = 128

    mesh = plsc.VectorSubcoreMesh(core_axis_name="c", subcore_axis_name="s")
    body = functools.partial(_densify_body, n, nnz, blk_rows)
    flat = pl.kernel(
        body,
        out_type=jax.ShapeDtypeStruct((n * n,), jnp.float32),
        mesh=mesh,
        scratch_types=[
            pltpu.VMEM((ept,), jnp.int32),
            pltpu.VMEM((ept,), jnp.int32),
            pltpu.VMEM((ept,), jnp.float32),
            pltpu.VMEM((ept // scat_w, scat_w), jnp.int32),
            pltpu.VMEM((ept // scat_w, scat_w), jnp.float32),
            pltpu.VMEM((blk_cells // _NS,), jnp.float32),
            pltpu.VMEM_SHARED((blk_cells,), jnp.float32),
        ],
    )(rows, cols, vals)
    return flat.reshape(n, n)


def _mm_body(a_ref, b_ref, bias_ref, o_ref, acc_ref):
    @pl.when(pl.program_id(2) == 0)
    def _():
        acc_ref[...] = jnp.zeros_like(acc_ref)

    acc_ref[...] += jnp.dot(a_ref[...], b_ref[...],
                            preferred_element_type=jnp.float32,
                            precision=jax.lax.Precision.HIGHEST)

    @pl.when(pl.program_id(2) == pl.num_programs(2) - 1)
    def _():
        o_ref[...] = acc_ref[...] + bias_ref[...]


def _mm_bias(a, b, bias2d, bm=512, bn=512, bk=1024):
    n = a.shape[0]
    grid = (n // bm, n // bn, n // bk)
    return pl.pallas_call(
        _mm_body,
        grid=grid,
        in_specs=[
            pl.BlockSpec((bm, bk), lambda i, j, k: (i, k)),
            pl.BlockSpec((bk, bn), lambda i, j, k: (k, j)),
            pl.BlockSpec((1, bn), lambda i, j, k: (0, j)),
        ],
        out_specs=pl.BlockSpec((bm, bn), lambda i, j, k: (i, j)),
        out_shape=jax.ShapeDtypeStruct((n, n), jnp.float32),
        scratch_shapes=[pltpu.VMEM((bm, bn), jnp.float32)],
        compiler_params=pltpu.CompilerParams(
            dimension_semantics=("parallel", "parallel", "arbitrary")),
    )(a, b, bias2d)


def kernel(x, rows, cols, a_vals, w_vals, bias):
    n = x.shape[0]
    w_d = _densify(rows, cols, w_vals, n)
    a_d = _densify(rows, cols, a_vals, n)
    zero_bias = jnp.zeros((1, n), jnp.float32)
    bias2d = bias.reshape(1, n)
    m = _mm_bias(a_d, w_d, zero_bias)
    out = _mm_bias(m, x, bias2d)
    out = _mm_bias(m, out, bias2d)
    return out


# speedup vs baseline: 4.9822x; 4.9822x over previous
import functools

import jax
import jax.numpy as jnp
from jax import lax
from jax.experimental import pallas as pl
from jax.experimental.pallas import tpu as pltpu
from jax.experimental.pallas import tpu_sc as plsc

_NC = 2
_NS = 16
_LANES = 16


def _densify_body(n, nnz, blk_rows, rows_hbm, cols_hbm, vals_hbm, out_hbm,
                  rows_v, cols_v, vals_v, offs_v, upds_v, zbuf_v, blk_sh):
    cid = lax.axis_index("c")
    sid = lax.axis_index("s")
    ept = nnz // _NS
    blk_cells = blk_rows * n
    cells_per_tec = blk_cells // _NS
    n_blocks = n // blk_rows
    blocks_per_core = n_blocks // _NC
    n_vregs = ept // _LANES
    scat_w = offs_v.shape[1]
    vregs_per_scat = scat_w // _LANES

    base_e = sid * ept
    pltpu.sync_copy(rows_hbm.at[pl.ds(base_e, ept)], rows_v)
    pltpu.sync_copy(cols_hbm.at[pl.ds(base_e, ept)], cols_v)
    pltpu.sync_copy(vals_hbm.at[pl.ds(base_e, ept)], vals_v)

    zchunk = zbuf_v.shape[0]
    def _z(i, _):
        zbuf_v[pl.ds(i * _LANES, _LANES)] = jnp.zeros((_LANES,), jnp.float32)
        return ()
    lax.fori_loop(0, zchunk // _LANES, _z, ())

    def _block(bi, _):
        blk = cid * blocks_per_core + bi
        base_r = blk * blk_rows

        def _clr(i, _):
            pltpu.sync_copy(
                zbuf_v,
                blk_sh.at[pl.ds(sid * cells_per_tec + i * zchunk, zchunk)])
            return ()
        lax.fori_loop(0, cells_per_tec // zchunk, _clr, ())
        plsc.subcore_barrier()

        def _prep(v, _):
            r = rows_v[pl.ds(v * _LANES, _LANES)]
            c = cols_v[pl.ds(v * _LANES, _LANES)]
            w = vals_v[pl.ds(v * _LANES, _LANES)]
            rel = r - base_r
            inblk = (rel >= 0) & (rel < blk_rows)
            dummy = sid * ept + v * _LANES + lax.iota(jnp.int32, _LANES)
            off = jnp.where(inblk, rel * n + c, dummy)
            val = jnp.where(inblk, w, jnp.zeros((_LANES,), jnp.float32))
            row = v // vregs_per_scat
            colo = (v % vregs_per_scat) * _LANES
            offs_v[row, pl.ds(colo, _LANES)] = off
            upds_v[row, pl.ds(colo, _LANES)] = val
            return ()
        lax.fori_loop(0, n_vregs, _prep, ())

        for j in range(ept // scat_w):
            pltpu.sync_copy(upds_v.at[j], blk_sh.at[offs_v.at[j]], add=True)
        plsc.subcore_barrier()

        pltpu.sync_copy(
            blk_sh.at[pl.ds(sid * cells_per_tec, cells_per_tec)],
            out_hbm.at[pl.ds(base_r * n + sid * cells_per_tec,
                             cells_per_tec)])
        plsc.subcore_barrier()
        return ()

    lax.fori_loop(0, blocks_per_core, _block, ())


def _densify(rows, cols, vals, n):
    nnz = rows.shape[0]
    blk_rows = 256
    blk_cells = blk_rows * n
    ept = nnz // _NS
    scat_w = 128

    mesh = plsc.VectorSubcoreMesh(core_axis_name="c", subcore_axis_name="s")
    body = functools.partial(_densify_body, n, nnz, blk_rows)
    flat = pl.kernel(
        body,
        out_type=jax.ShapeDtypeStruct((n * n,), jnp.float32),
        mesh=mesh,
        scratch_types=[
            pltpu.VMEM((ept,), jnp.int32),
            pltpu.VMEM((ept,), jnp.int32),
            pltpu.VMEM((ept,), jnp.float32),
            pltpu.VMEM((ept // scat_w, scat_w), jnp.int32),
            pltpu.VMEM((ept // scat_w, scat_w), jnp.float32),
            pltpu.VMEM((4096,), jnp.float32),
            pltpu.VMEM_SHARED((blk_cells,), jnp.float32),
        ],
    )(rows, cols, vals)
    return flat.reshape(n, n)


def _mm_body(a_ref, b_ref, bias_ref, o_ref, acc_ref):
    @pl.when(pl.program_id(2) == 0)
    def _():
        acc_ref[...] = jnp.zeros_like(acc_ref)

    acc_ref[...] += jnp.dot(a_ref[...], b_ref[...],
                            preferred_element_type=jnp.float32,
                            precision=jax.lax.Precision.HIGHEST)

    @pl.when(pl.program_id(2) == pl.num_programs(2) - 1)
    def _():
        o_ref[...] = acc_ref[...] + bias_ref[...]


def _mm_bias(a, b, bias2d, bm=512, bn=512, bk=1024):
    n = a.shape[0]
    grid = (n // bm, n // bn, n // bk)
    return pl.pallas_call(
        _mm_body,
        grid=grid,
        in_specs=[
            pl.BlockSpec((bm, bk), lambda i, j, k: (i, k)),
            pl.BlockSpec((bk, bn), lambda i, j, k: (k, j)),
            pl.BlockSpec((1, bn), lambda i, j, k: (0, j)),
        ],
        out_specs=pl.BlockSpec((bm, bn), lambda i, j, k: (i, j)),
        out_shape=jax.ShapeDtypeStruct((n, n), jnp.float32),
        scratch_shapes=[pltpu.VMEM((bm, bn), jnp.float32)],
        compiler_params=pltpu.CompilerParams(
            dimension_semantics=("parallel", "parallel", "arbitrary")),
    )(a, b, bias2d)


def kernel(x, rows, cols, a_vals, w_vals, bias):
    n = x.shape[0]
    w_d = _densify(rows, cols, w_vals, n)
    a_d = _densify(rows, cols, a_vals, n)
    zero_bias = jnp.zeros((1, n), jnp.float32)
    bias2d = bias.reshape(1, n)
    m = _mm_bias(a_d, w_d, zero_bias)
    out = _mm_bias(m, x, bias2d)
    out = _mm_bias(m, out, bias2d)
    return out


# speedup vs baseline: 7.4267x; 1.4906x over previous
import functools

import jax
import jax.numpy as jnp
from jax import lax
from jax.experimental import pallas as pl
from jax.experimental.pallas import tpu as pltpu
from jax.experimental.pallas import tpu_sc as plsc

_NC = 2
_NS = 16
_LANES = 16


def _densify_body(n, nnz, blk_rows, rows_hbm, cols_hbm, wvals_hbm, avals_hbm,
                  outw_hbm, outa_hbm,
                  rows_v, cols_v, wv_v, av_v, offs_v, upds_v, zbuf_v, blk_sh):
    cid = lax.axis_index("c")
    sid = lax.axis_index("s")
    ept = nnz // _NS
    blk_cells = blk_rows * n
    cells_per_tec = blk_cells // _NS
    n_blocks = n // blk_rows
    n_vregs = ept // _LANES
    scat_w = offs_v.shape[1]
    vregs_per_scat = scat_w // _LANES

    base_e = sid * ept
    pltpu.sync_copy(rows_hbm.at[pl.ds(base_e, ept)], rows_v)
    pltpu.sync_copy(cols_hbm.at[pl.ds(base_e, ept)], cols_v)
    pltpu.sync_copy(wvals_hbm.at[pl.ds(base_e, ept)], wv_v)
    pltpu.sync_copy(avals_hbm.at[pl.ds(base_e, ept)], av_v)

    zchunk = zbuf_v.shape[0]
    def _z(i, _):
        zbuf_v[pl.ds(i * _LANES, _LANES)] = jnp.zeros((_LANES,), jnp.float32)
        return ()
    lax.fori_loop(0, zchunk // _LANES, _z, ())

    def _block(bi, _):
        base_r = bi * blk_rows

        def _clr(i, _):
            pltpu.sync_copy(
                zbuf_v,
                blk_sh.at[pl.ds(sid * cells_per_tec + i * zchunk, zchunk)])
            return ()
        lax.fori_loop(0, cells_per_tec // zchunk, _clr, ())
        plsc.subcore_barrier()

        def _prep(v, _):
            r = rows_v[pl.ds(v * _LANES, _LANES)]
            c = cols_v[pl.ds(v * _LANES, _LANES)]
            wv = wv_v[pl.ds(v * _LANES, _LANES)]
            av = av_v[pl.ds(v * _LANES, _LANES)]
            w = jnp.where(cid == 0, wv, av)
            rel = r - base_r
            inblk = (rel >= 0) & (rel < blk_rows)
            dummy = sid * ept + v * _LANES + lax.iota(jnp.int32, _LANES)
            off = jnp.where(inblk, rel * n + c, dummy)
            val = jnp.where(inblk, w, jnp.zeros((_LANES,), jnp.float32))
            row = v // vregs_per_scat
            colo = (v % vregs_per_scat) * _LANES
            offs_v[row, pl.ds(colo, _LANES)] = off
            upds_v[row, pl.ds(colo, _LANES)] = val
            return ()
        lax.fori_loop(0, n_vregs, _prep, ())

        for j in range(ept // scat_w):
            pltpu.sync_copy(upds_v.at[j], blk_sh.at[offs_v.at[j]], add=True)
        plsc.subcore_barrier()

        @pl.when(cid == 0)
        def _():
            pltpu.sync_copy(
                blk_sh.at[pl.ds(sid * cells_per_tec, cells_per_tec)],
                outw_hbm.at[pl.ds(base_r * n + sid * cells_per_tec,
                                  cells_per_tec)])

        @pl.when(cid == 1)
        def _():
            pltpu.sync_copy(
                blk_sh.at[pl.ds(sid * cells_per_tec, cells_per_tec)],
                outa_hbm.at[pl.ds(base_r * n + sid * cells_per_tec,
                                  cells_per_tec)])
        plsc.subcore_barrier()
        return ()

    lax.fori_loop(0, n_blocks, _block, ())


def _densify(rows, cols, w_vals, a_vals, n):
    nnz = rows.shape[0]
    blk_rows = 256
    blk_cells = blk_rows * n
    ept = nnz // _NS
    scat_w = 128

    mesh = plsc.VectorSubcoreMesh(core_axis_name="c", subcore_axis_name="s")
    body = functools.partial(_densify_body, n, nnz, blk_rows)
    flat_w, flat_a = pl.kernel(
        body,
        out_type=(jax.ShapeDtypeStruct((n * n,), jnp.float32),
                  jax.ShapeDtypeStruct((n * n,), jnp.float32)),
        mesh=mesh,
        scratch_types=[
            pltpu.VMEM((ept,), jnp.int32),
            pltpu.VMEM((ept,), jnp.int32),
            pltpu.VMEM((ept,), jnp.float32),
            pltpu.VMEM((ept,), jnp.float32),
            pltpu.VMEM((ept // scat_w, scat_w), jnp.int32),
            pltpu.VMEM((ept // scat_w, scat_w), jnp.float32),
            pltpu.VMEM((4096,), jnp.float32),
            pltpu.VMEM_SHARED((blk_cells,), jnp.float32),
        ],
    )(rows, cols, w_vals, a_vals)
    return flat_w.reshape(n, n), flat_a.reshape(n, n)


def _mm_body(a_ref, b_ref, bias_ref, o_ref, acc_ref):
    @pl.when(pl.program_id(2) == 0)
    def _():
        acc_ref[...] = jnp.zeros_like(acc_ref)

    a = a_ref[...]
    b = b_ref[...]
    ah = a.astype(jnp.bfloat16)
    al = (a - ah.astype(jnp.float32)).astype(jnp.bfloat16)
    bh = b.astype(jnp.bfloat16)
    bl = (b - bh.astype(jnp.float32)).astype(jnp.bfloat16)
    acc_ref[...] += (
        jnp.dot(ah, bh, preferred_element_type=jnp.float32)
        + jnp.dot(ah, bl, preferred_element_type=jnp.float32)
        + jnp.dot(al, bh, preferred_element_type=jnp.float32))

    @pl.when(pl.program_id(2) == pl.num_programs(2) - 1)
    def _():
        o_ref[...] = acc_ref[...] + bias_ref[...]


def _mm_bias(a, b, bias2d, bm=512, bn=512, bk=1024):
    n = a.shape[0]
    grid = (n // bm, n // bn, n // bk)
    return pl.pallas_call(
        _mm_body,
        grid=grid,
        in_specs=[
            pl.BlockSpec((bm, bk), lambda i, j, k: (i, k)),
            pl.BlockSpec((bk, bn), lambda i, j, k: (k, j)),
            pl.BlockSpec((1, bn), lambda i, j, k: (0, j)),
        ],
        out_specs=pl.BlockSpec((bm, bn), lambda i, j, k: (i, j)),
        out_shape=jax.ShapeDtypeStruct((n, n), jnp.float32),
        scratch_shapes=[pltpu.VMEM((bm, bn), jnp.float32)],
        compiler_params=pltpu.CompilerParams(
            dimension_semantics=("parallel", "parallel", "arbitrary")),
    )(a, b, bias2d)


def kernel(x, rows, cols, a_vals, w_vals, bias):
    n = x.shape[0]
    w_d, a_d = _densify(rows, cols, w_vals, a_vals, n)
    zero_bias = jnp.zeros((1, n), jnp.float32)
    bias2d = bias.reshape(1, n)
    m = _mm_bias(a_d, w_d, zero_bias)
    out = _mm_bias(m, x, bias2d)
    out = _mm_bias(m, out, bias2d)
    return out


# speedup vs baseline: 9.0801x; 1.2226x over previous
import functools

import jax
import jax.numpy as jnp
from jax import lax
from jax.experimental import pallas as pl
from jax.experimental.pallas import tpu as pltpu
from jax.experimental.pallas import tpu_sc as plsc

_NC = 2
_NS = 16
_LANES = 16


def _densify_body(n, nnz, blk_rows, rows_hbm, cols_hbm, wvals_hbm, avals_hbm,
                  outw_hbm, outa_hbm,
                  rows_v, cols_v, wv_v, av_v, offs_v, upds_v, zbuf_v, blk_sh):
    cid = lax.axis_index("c")
    sid = lax.axis_index("s")
    ept = nnz // _NS
    blk_cells = blk_rows * n
    cells_per_tec = blk_cells // _NS
    n_blocks = n // blk_rows
    n_vregs = ept // _LANES
    scat_w = offs_v.shape[1]
    vregs_per_scat = scat_w // _LANES

    base_e = sid * ept
    pltpu.sync_copy(rows_hbm.at[pl.ds(base_e, ept)], rows_v)
    pltpu.sync_copy(cols_hbm.at[pl.ds(base_e, ept)], cols_v)
    pltpu.sync_copy(wvals_hbm.at[pl.ds(base_e, ept)], wv_v)
    pltpu.sync_copy(avals_hbm.at[pl.ds(base_e, ept)], av_v)

    zchunk = zbuf_v.shape[0]
    def _z(i, _):
        zbuf_v[pl.ds(i * _LANES, _LANES)] = jnp.zeros((_LANES,), jnp.float32)
        return ()
    lax.fori_loop(0, zchunk // _LANES, _z, ())

    def _block(bi, _):
        base_r = bi * blk_rows

        def _clr(i, _):
            pltpu.sync_copy(
                zbuf_v,
                blk_sh.at[pl.ds(sid * cells_per_tec + i * zchunk, zchunk)])
            return ()
        lax.fori_loop(0, cells_per_tec // zchunk, _clr, ())
        plsc.subcore_barrier()

        def _prep(v, _):
            r = rows_v[pl.ds(v * _LANES, _LANES)]
            c = cols_v[pl.ds(v * _LANES, _LANES)]
            wv = wv_v[pl.ds(v * _LANES, _LANES)]
            av = av_v[pl.ds(v * _LANES, _LANES)]
            w = jnp.where(cid == 0, wv, av)
            rel = r - base_r
            inblk = (rel >= 0) & (rel < blk_rows)
            dummy = sid * ept + v * _LANES + lax.iota(jnp.int32, _LANES)
            off = jnp.where(inblk, rel * n + c, dummy)
            val = jnp.where(inblk, w, jnp.zeros((_LANES,), jnp.float32))
            row = v // vregs_per_scat
            colo = (v % vregs_per_scat) * _LANES
            offs_v[row, pl.ds(colo, _LANES)] = off
            upds_v[row, pl.ds(colo, _LANES)] = val
            return ()
        lax.fori_loop(0, n_vregs, _prep, ())

        for j in range(ept // scat_w):
            pltpu.sync_copy(upds_v.at[j], blk_sh.at[offs_v.at[j]], add=True)
        plsc.subcore_barrier()

        @pl.when(cid == 0)
        def _():
            pltpu.sync_copy(
                blk_sh.at[pl.ds(sid * cells_per_tec, cells_per_tec)],
                outw_hbm.at[pl.ds(base_r * n + sid * cells_per_tec,
                                  cells_per_tec)])

        @pl.when(cid == 1)
        def _():
            pltpu.sync_copy(
                blk_sh.at[pl.ds(sid * cells_per_tec, cells_per_tec)],
                outa_hbm.at[pl.ds(base_r * n + sid * cells_per_tec,
                                  cells_per_tec)])
        plsc.subcore_barrier()
        return ()

    lax.fori_loop(0, n_blocks, _block, ())


def _densify(rows, cols, w_vals, a_vals, n):
    nnz = rows.shape[0]
    blk_rows = 256
    blk_cells = blk_rows * n
    ept = nnz // _NS
    scat_w = 128

    mesh = plsc.VectorSubcoreMesh(core_axis_name="c", subcore_axis_name="s")
    body = functools.partial(_densify_body, n, nnz, blk_rows)
    flat_w, flat_a = pl.kernel(
        body,
        out_type=(jax.ShapeDtypeStruct((n * n,), jnp.float32),
                  jax.ShapeDtypeStruct((n * n,), jnp.float32)),
        mesh=mesh,
        scratch_types=[
            pltpu.VMEM((ept,), jnp.int32),
            pltpu.VMEM((ept,), jnp.int32),
            pltpu.VMEM((ept,), jnp.float32),
            pltpu.VMEM((ept,), jnp.float32),
            pltpu.VMEM((ept // scat_w, scat_w), jnp.int32),
            pltpu.VMEM((ept // scat_w, scat_w), jnp.float32),
            pltpu.VMEM((4096,), jnp.float32),
            pltpu.VMEM_SHARED((blk_cells,), jnp.float32),
        ],
    )(rows, cols, w_vals, a_vals)
    return flat_w.reshape(n, n), flat_a.reshape(n, n)


def _mm_body(a_ref, b_ref, bias_ref, o_ref, acc_ref):
    @pl.when(pl.program_id(2) == 0)
    def _():
        acc_ref[...] = jnp.zeros_like(acc_ref)

    a = a_ref[...]
    b = b_ref[...]
    ah = a.astype(jnp.bfloat16)
    al = (a - ah.astype(jnp.float32)).astype(jnp.bfloat16)
    bh = b.astype(jnp.bfloat16)
    bl = (b - bh.astype(jnp.float32)).astype(jnp.bfloat16)
    acc_ref[...] += (
        jnp.dot(ah, bh, preferred_element_type=jnp.float32)
        + jnp.dot(ah, bl, preferred_element_type=jnp.float32)
        + jnp.dot(al, bh, preferred_element_type=jnp.float32))

    @pl.when(pl.program_id(2) == pl.num_programs(2) - 1)
    def _():
        o_ref[...] = acc_ref[...] + bias_ref[...]


def _mm_bias(a, b, bias2d, bm=1024, bn=1024, bk=2048):
    n = a.shape[0]
    grid = (n // bm, n // bn, n // bk)
    return pl.pallas_call(
        _mm_body,
        grid=grid,
        in_specs=[
            pl.BlockSpec((bm, bk), lambda i, j, k: (i, k)),
            pl.BlockSpec((bk, bn), lambda i, j, k: (k, j)),
            pl.BlockSpec((1, bn), lambda i, j, k: (0, j)),
        ],
        out_specs=pl.BlockSpec((bm, bn), lambda i, j, k: (i, j)),
        out_shape=jax.ShapeDtypeStruct((n, n), jnp.float32),
        scratch_shapes=[pltpu.VMEM((bm, bn), jnp.float32)],
        compiler_params=pltpu.CompilerParams(
            dimension_semantics=("parallel", "parallel", "arbitrary")),
    )(a, b, bias2d)


def kernel(x, rows, cols, a_vals, w_vals, bias):
    n = x.shape[0]
    w_d, a_d = _densify(rows, cols, w_vals, a_vals, n)
    zero_bias = jnp.zeros((1, n), jnp.float32)
    bias2d = bias.reshape(1, n)
    m = _mm_bias(a_d, w_d, zero_bias)
    out = _mm_bias(m, x, bias2d)
    out = _mm_bias(m, out, bias2d)
    return out


# speedup vs baseline: 15.9448x; 1.7560x over previous
import functools

import jax
import jax.numpy as jnp
from jax import lax
from jax.experimental import pallas as pl
from jax.experimental.pallas import tpu as pltpu
from jax.experimental.pallas import tpu_sc as plsc

_NC = 2
_NS = 16
_LANES = 16


def _densify_body(n, nnz, blk_rows, rows_hbm, cols_hbm, wvals_hbm, avals_hbm,
                  outw_hbm, outa_hbm,
                  rows_v, cols_v, wv_v, av_v, offs_v, upds_v, zbuf_v, blk_sh):
    cid = lax.axis_index("c")
    sid = lax.axis_index("s")
    ept = nnz // _NS
    blk_cells = blk_rows * n
    cells_per_tec = blk_cells // _NS
    n_blocks = n // blk_rows
    n_vregs = ept // _LANES
    scat_w = offs_v.shape[1]
    vregs_per_scat = scat_w // _LANES

    base_e = sid * ept
    pltpu.sync_copy(rows_hbm.at[pl.ds(base_e, ept)], rows_v)
    pltpu.sync_copy(cols_hbm.at[pl.ds(base_e, ept)], cols_v)
    pltpu.sync_copy(wvals_hbm.at[pl.ds(base_e, ept)], wv_v)
    pltpu.sync_copy(avals_hbm.at[pl.ds(base_e, ept)], av_v)

    zchunk = zbuf_v.shape[0]
    def _z(i, _):
        zbuf_v[pl.ds(i * _LANES, _LANES)] = jnp.zeros((_LANES,), jnp.float32)
        return ()
    lax.fori_loop(0, zchunk // _LANES, _z, ())

    def _block(bi, _):
        base_r = bi * blk_rows

        def _clr(i, _):
            pltpu.sync_copy(
                zbuf_v,
                blk_sh.at[pl.ds(sid * cells_per_tec + i * zchunk, zchunk)])
            return ()
        lax.fori_loop(0, cells_per_tec // zchunk, _clr, ())
        plsc.subcore_barrier()

        def _prep(v, _):
            r = rows_v[pl.ds(v * _LANES, _LANES)]
            c = cols_v[pl.ds(v * _LANES, _LANES)]
            wv = wv_v[pl.ds(v * _LANES, _LANES)]
            av = av_v[pl.ds(v * _LANES, _LANES)]
            w = jnp.where(cid == 0, wv, av)
            rel = r - base_r
            inblk = (rel >= 0) & (rel < blk_rows)
            dummy = sid * ept + v * _LANES + lax.iota(jnp.int32, _LANES)
            off = jnp.where(inblk, rel * n + c, dummy)
            val = jnp.where(inblk, w, jnp.zeros((_LANES,), jnp.float32))
            row = v // vregs_per_scat
            colo = (v % vregs_per_scat) * _LANES
            offs_v[row, pl.ds(colo, _LANES)] = off
            upds_v[row, pl.ds(colo, _LANES)] = val
            return ()
        lax.fori_loop(0, n_vregs, _prep, ())

        for j in range(ept // scat_w):
            pltpu.sync_copy(upds_v.at[j], blk_sh.at[offs_v.at[j]], add=True)
        plsc.subcore_barrier()

        @pl.when(cid == 0)
        def _():
            pltpu.sync_copy(
                blk_sh.at[pl.ds(sid * cells_per_tec, cells_per_tec)],
                outw_hbm.at[pl.ds(base_r * n + sid * cells_per_tec,
                                  cells_per_tec)])

        @pl.when(cid == 1)
        def _():
            pltpu.sync_copy(
                blk_sh.at[pl.ds(sid * cells_per_tec, cells_per_tec)],
                outa_hbm.at[pl.ds(base_r * n + sid * cells_per_tec,
                                  cells_per_tec)])
        plsc.subcore_barrier()
        return ()

    lax.fori_loop(0, n_blocks, _block, ())


def _densify(rows, cols, w_vals, a_vals, n):
    nnz = rows.shape[0]
    blk_rows = 256
    blk_cells = blk_rows * n
    ept = nnz // _NS
    scat_w = 128

    mesh = plsc.VectorSubcoreMesh(core_axis_name="c", subcore_axis_name="s")
    body = functools.partial(_densify_body, n, nnz, blk_rows)
    flat_w, flat_a = pl.kernel(
        body,
        out_type=(jax.ShapeDtypeStruct((n * n,), jnp.float32),
                  jax.ShapeDtypeStruct((n * n,), jnp.float32)),
        mesh=mesh,
        scratch_types=[
            pltpu.VMEM((ept,), jnp.int32),
            pltpu.VMEM((ept,), jnp.int32),
            pltpu.VMEM((ept,), jnp.float32),
            pltpu.VMEM((ept,), jnp.float32),
            pltpu.VMEM((ept // scat_w, scat_w), jnp.int32),
            pltpu.VMEM((ept // scat_w, scat_w), jnp.float32),
            pltpu.VMEM((4096,), jnp.float32),
            pltpu.VMEM_SHARED((blk_cells,), jnp.float32),
        ],
    )(rows, cols, w_vals, a_vals)
    return flat_w.reshape(n, n), flat_a.reshape(n, n)


def _mm_body(a_ref, b_ref, bias_ref, o_ref, acc_ref):
    @pl.when(pl.program_id(2) == 0)
    def _():
        acc_ref[...] = jnp.zeros_like(acc_ref)

    ah = a_ref[...].astype(jnp.bfloat16)
    bh = b_ref[...].astype(jnp.bfloat16)
    acc_ref[...] += jnp.dot(ah, bh, preferred_element_type=jnp.float32)

    @pl.when(pl.program_id(2) == pl.num_programs(2) - 1)
    def _():
        o_ref[...] = acc_ref[...] + bias_ref[...]


def _mm_bias(a, b, bias2d, bm=1024, bn=1024, bk=2048):
    n = a.shape[0]
    grid = (n // bm, n // bn, n // bk)
    return pl.pallas_call(
        _mm_body,
        grid=grid,
        in_specs=[
            pl.BlockSpec((bm, bk), lambda i, j, k: (i, k)),
            pl.BlockSpec((bk, bn), lambda i, j, k: (k, j)),
            pl.BlockSpec((1, bn), lambda i, j, k: (0, j)),
        ],
        out_specs=pl.BlockSpec((bm, bn), lambda i, j, k: (i, j)),
        out_shape=jax.ShapeDtypeStruct((n, n), jnp.float32),
        scratch_shapes=[pltpu.VMEM((bm, bn), jnp.float32)],
        compiler_params=pltpu.CompilerParams(
            dimension_semantics=("parallel", "parallel", "arbitrary")),
    )(a, b, bias2d)


def kernel(x, rows, cols, a_vals, w_vals, bias):
    n = x.shape[0]
    w_d, a_d = _densify(rows, cols, w_vals, a_vals, n)
    zero_bias = jnp.zeros((1, n), jnp.float32)
    bias2d = bias.reshape(1, n)
    m = _mm_bias(a_d, w_d, zero_bias)
    out = _mm_bias(m, x, bias2d)
    out = _mm_bias(m, out, bias2d)
    return out


# speedup vs baseline: 17.9034x; 1.1228x over previous
import functools

import jax
import jax.numpy as jnp
from jax import lax
from jax.experimental import pallas as pl
from jax.experimental.pallas import tpu as pltpu
from jax.experimental.pallas import tpu_sc as plsc

_NC = 2
_NS = 16
_LANES = 16


def _densify_body(n, nnz, blk_rows, rows_hbm, cols_hbm, wvals_hbm, avals_hbm,
                  outw_hbm, outa_hbm,
                  rows_v, cols_v, wv_v, av_v, offs_v, upds_v, zbuf_v, blk_sh,
                  sem_c, sem_s, sem_w0, sem_w1):
    cid = lax.axis_index("c")
    sid = lax.axis_index("s")
    ept = nnz // _NS
    blk_cells = blk_rows * n
    cells_per_tec = blk_cells // _NS
    n_blocks = n // blk_rows
    n_vregs = ept // _LANES
    scat_w = offs_v.shape[1]
    vregs_per_scat = scat_w // _LANES

    base_e = sid * ept
    pltpu.sync_copy(rows_hbm.at[pl.ds(base_e, ept)], rows_v)
    pltpu.sync_copy(cols_hbm.at[pl.ds(base_e, ept)], cols_v)
    pltpu.sync_copy(wvals_hbm.at[pl.ds(base_e, ept)], wv_v)
    pltpu.sync_copy(avals_hbm.at[pl.ds(base_e, ept)], av_v)

    zchunk = zbuf_v.shape[0]
    def _z(i, _):
        zbuf_v[pl.ds(i * _LANES, _LANES)] = jnp.zeros((_LANES,), jnp.float32)
        return ()
    lax.fori_loop(0, zchunk // _LANES, _z, ())

    def _out_slice(base_r, out_hbm):
        return out_hbm.at[pl.ds(base_r * n + sid * cells_per_tec,
                                cells_per_tec)]

    def _buf_slice(par):
        return blk_sh.at[pl.ds(par * blk_cells + sid * cells_per_tec,
                               cells_per_tec)]

    def _wait_writeback(par, base_r_prev):
        @pl.when(cid == 0)
        def _():
            @pl.when(par == 0)
            def _():
                pltpu.make_async_copy(
                    _buf_slice(par), _out_slice(base_r_prev, outw_hbm),
                    sem_w0).wait()
            @pl.when(par == 1)
            def _():
                pltpu.make_async_copy(
                    _buf_slice(par), _out_slice(base_r_prev, outw_hbm),
                    sem_w1).wait()
        @pl.when(cid == 1)
        def _():
            @pl.when(par == 0)
            def _():
                pltpu.make_async_copy(
                    _buf_slice(par), _out_slice(base_r_prev, outa_hbm),
                    sem_w0).wait()
            @pl.when(par == 1)
            def _():
                pltpu.make_async_copy(
                    _buf_slice(par), _out_slice(base_r_prev, outa_hbm),
                    sem_w1).wait()

    def _block(bi, _):
        base_r = bi * blk_rows
        par = lax.rem(bi, 2)
        buf_base = par * blk_cells

        @pl.when(bi >= 2)
        def _():
            _wait_writeback(par, base_r - 2 * blk_rows)

        clrs = []
        for i in range(cells_per_tec // zchunk):
            clrs.append(pltpu.async_copy(
                zbuf_v,
                blk_sh.at[pl.ds(buf_base + sid * cells_per_tec + i * zchunk,
                                zchunk)],
                sem_c))

        def _prep(v, _):
            r = rows_v[pl.ds(v * _LANES, _LANES)]
            c = cols_v[pl.ds(v * _LANES, _LANES)]
            wv = wv_v[pl.ds(v * _LANES, _LANES)]
            av = av_v[pl.ds(v * _LANES, _LANES)]
            w = jnp.where(cid == 0, wv, av)
            rel = r - base_r
            inblk = (rel >= 0) & (rel < blk_rows)
            dummy = sid * ept + v * _LANES + lax.iota(jnp.int32, _LANES)
            off = buf_base + jnp.where(inblk, rel * n + c, dummy)
            val = jnp.where(inblk, w, jnp.zeros((_LANES,), jnp.float32))
            row = v // vregs_per_scat
            colo = (v % vregs_per_scat) * _LANES
            offs_v[row, pl.ds(colo, _LANES)] = off
            upds_v[row, pl.ds(colo, _LANES)] = val
            return ()
        lax.fori_loop(0, n_vregs, _prep, ())
        for d in clrs:
            d.wait()
        plsc.subcore_barrier()

        scats = []
        for j in range(ept // scat_w):
            scats.append(pltpu.async_copy(
                upds_v.at[j], blk_sh.at[offs_v.at[j]], sem_s, add=True))
        for d in scats:
            d.wait()
        plsc.subcore_barrier()

        @pl.when(cid == 0)
        def _():
            @pl.when(par == 0)
            def _():
                pltpu.async_copy(_buf_slice(par),
                                 _out_slice(base_r, outw_hbm), sem_w0)
            @pl.when(par == 1)
            def _():
                pltpu.async_copy(_buf_slice(par),
                                 _out_slice(base_r, outw_hbm), sem_w1)
        @pl.when(cid == 1)
        def _():
            @pl.when(par == 0)
            def _():
                pltpu.async_copy(_buf_slice(par),
                                 _out_slice(base_r, outa_hbm), sem_w0)
            @pl.when(par == 1)
            def _():
                pltpu.async_copy(_buf_slice(par),
                                 _out_slice(base_r, outa_hbm), sem_w1)
        return ()

    lax.fori_loop(0, n_blocks, _block, ())

    _wait_writeback(jnp.int32(0), (n_blocks - 2) * blk_rows)
    _wait_writeback(jnp.int32(1), (n_blocks - 1) * blk_rows)


def _densify(rows, cols, w_vals, a_vals, n):
    nnz = rows.shape[0]
    blk_rows = 128
    blk_cells = blk_rows * n
    ept = nnz // _NS
    scat_w = 128

    mesh = plsc.VectorSubcoreMesh(core_axis_name="c", subcore_axis_name="s")
    body = functools.partial(_densify_body, n, nnz, blk_rows)
    flat_w, flat_a = pl.kernel(
        body,
        out_type=(jax.ShapeDtypeStruct((n * n,), jnp.float32),
                  jax.ShapeDtypeStruct((n * n,), jnp.float32)),
        mesh=mesh,
        scratch_types=[
            pltpu.VMEM((ept,), jnp.int32),
            pltpu.VMEM((ept,), jnp.int32),
            pltpu.VMEM((ept,), jnp.float32),
            pltpu.VMEM((ept,), jnp.float32),
            pltpu.VMEM((ept // scat_w, scat_w), jnp.int32),
            pltpu.VMEM((ept // scat_w, scat_w), jnp.float32),
            pltpu.VMEM((16384,), jnp.float32),
            pltpu.VMEM_SHARED((2 * blk_cells,), jnp.float32),
            pltpu.SemaphoreType.DMA,
            pltpu.SemaphoreType.DMA,
            pltpu.SemaphoreType.DMA,
            pltpu.SemaphoreType.DMA,
        ],
    )(rows, cols, w_vals, a_vals)
    return flat_w.reshape(n, n), flat_a.reshape(n, n)


def _mm_body(a_ref, b_ref, bias_ref, o_ref, acc_ref):
    @pl.when(pl.program_id(2) == 0)
    def _():
        acc_ref[...] = jnp.zeros_like(acc_ref)

    ah = a_ref[...].astype(jnp.bfloat16)
    bh = b_ref[...].astype(jnp.bfloat16)
    acc_ref[...] += jnp.dot(ah, bh, preferred_element_type=jnp.float32)

    @pl.when(pl.program_id(2) == pl.num_programs(2) - 1)
    def _():
        o_ref[...] = acc_ref[...] + bias_ref[...]


def _mm_bias(a, b, bias2d, bm=1024, bn=1024, bk=2048):
    n = a.shape[0]
    grid = (n // bm, n // bn, n // bk)
    return pl.pallas_call(
        _mm_body,
        grid=grid,
        in_specs=[
            pl.BlockSpec((bm, bk), lambda i, j, k: (i, k)),
            pl.BlockSpec((bk, bn), lambda i, j, k: (k, j)),
            pl.BlockSpec((1, bn), lambda i, j, k: (0, j)),
        ],
        out_specs=pl.BlockSpec((bm, bn), lambda i, j, k: (i, j)),
        out_shape=jax.ShapeDtypeStruct((n, n), jnp.float32),
        scratch_shapes=[pltpu.VMEM((bm, bn), jnp.float32)],
        compiler_params=pltpu.CompilerParams(
            dimension_semantics=("parallel", "parallel", "arbitrary")),
    )(a, b, bias2d)


def kernel(x, rows, cols, a_vals, w_vals, bias):
    n = x.shape[0]
    w_d, a_d = _densify(rows, cols, w_vals, a_vals, n)
    zero_bias = jnp.zeros((1, n), jnp.float32)
    bias2d = bias.reshape(1, n)
    m = _mm_bias(a_d, w_d, zero_bias)
    out = _mm_bias(m, x, bias2d)
    out = _mm_bias(m, out, bias2d)
    return out


# speedup vs baseline: 19.4867x; 1.0884x over previous
import functools

import jax
import jax.numpy as jnp
from jax import lax
from jax.experimental import pallas as pl
from jax.experimental.pallas import tpu as pltpu
from jax.experimental.pallas import tpu_sc as plsc

_NC = 2
_NS = 16
_LANES = 16


def _densify_body(n, nnz, blk_rows, rows_hbm, cols_hbm, wvals_hbm, avals_hbm,
                  outw_hbm, outa_hbm,
                  rows_v, cols_v, wv_v, av_v, offs_v, upds_v, zbuf_v, blk_sh,
                  sem_c, sem_s, sem_w0, sem_w1):
    cid = lax.axis_index("c")
    sid = lax.axis_index("s")
    ept = nnz // _NS
    blk_cells = blk_rows * n
    cells_per_tec = blk_cells // _NS
    n_blocks = n // blk_rows
    n_vregs = ept // _LANES
    scat_w = offs_v.shape[1]
    vregs_per_scat = scat_w // _LANES

    base_e = sid * ept
    pltpu.sync_copy(rows_hbm.at[pl.ds(base_e, ept)], rows_v)
    pltpu.sync_copy(cols_hbm.at[pl.ds(base_e, ept)], cols_v)
    pltpu.sync_copy(wvals_hbm.at[pl.ds(base_e, ept)], wv_v)
    pltpu.sync_copy(avals_hbm.at[pl.ds(base_e, ept)], av_v)

    zchunk = zbuf_v.shape[0]
    def _z(i, _):
        zbuf_v[pl.ds(i * _LANES, _LANES)] = jnp.zeros((_LANES,), jnp.float32)
        return ()
    lax.fori_loop(0, zchunk // _LANES, _z, ())

    rows_per_tec = blk_rows // _NS

    def _out_slice(base_r, out_hbm):
        return _out_flat_slice(base_r, out_hbm)

    def _out_flat_slice(base_r, out_hbm):
        return out_hbm.at[pl.ds(base_r * n + sid * cells_per_tec,
                                cells_per_tec)]

    def _buf_slice(par):
        return blk_sh.at[pl.ds(par * blk_cells + sid * cells_per_tec,
                               cells_per_tec)]

    def _wait_writeback(par, base_r_prev):
        @pl.when(cid == 0)
        def _():
            @pl.when(par == 0)
            def _():
                pltpu.make_async_copy(
                    _buf_slice(par), _out_slice(base_r_prev, outw_hbm),
                    sem_w0).wait()
            @pl.when(par == 1)
            def _():
                pltpu.make_async_copy(
                    _buf_slice(par), _out_slice(base_r_prev, outw_hbm),
                    sem_w1).wait()
        @pl.when(cid == 1)
        def _():
            @pl.when(par == 0)
            def _():
                pltpu.make_async_copy(
                    _buf_slice(par), _out_slice(base_r_prev, outa_hbm),
                    sem_w0).wait()
            @pl.when(par == 1)
            def _():
                pltpu.make_async_copy(
                    _buf_slice(par), _out_slice(base_r_prev, outa_hbm),
                    sem_w1).wait()

    def _block(bi, _):
        base_r = bi * blk_rows
        par = lax.rem(bi, 2)
        buf_base = par * blk_cells

        @pl.when(bi >= 2)
        def _():
            _wait_writeback(par, base_r - 2 * blk_rows)

        clrs = []
        for i in range(cells_per_tec // zchunk):
            clrs.append(pltpu.async_copy(
                zbuf_v,
                blk_sh.at[pl.ds(buf_base + sid * cells_per_tec + i * zchunk,
                                zchunk)],
                sem_c))

        def _prep(v, _):
            r = rows_v[pl.ds(v * _LANES, _LANES)]
            c = cols_v[pl.ds(v * _LANES, _LANES)]
            wv = wv_v[pl.ds(v * _LANES, _LANES)]
            av = av_v[pl.ds(v * _LANES, _LANES)]
            w = jnp.where(cid == 0, wv, av)
            rel = r - base_r
            inblk = (rel >= 0) & (rel < blk_rows)
            dummy = sid * ept + v * _LANES + lax.iota(jnp.int32, _LANES)
            off = buf_base + jnp.where(inblk, rel * n + c, dummy)
            val = jnp.where(inblk, w, jnp.zeros((_LANES,), jnp.float32))
            row = v // vregs_per_scat
            colo = (v % vregs_per_scat) * _LANES
            offs_v[row, pl.ds(colo, _LANES)] = off
            upds_v[row, pl.ds(colo, _LANES)] = val
            return ()
        lax.fori_loop(0, n_vregs, _prep, ())
        for d in clrs:
            d.wait()
        plsc.subcore_barrier()

        scats = []
        for j in range(ept // scat_w):
            scats.append(pltpu.async_copy(
                upds_v.at[j], blk_sh.at[offs_v.at[j]], sem_s, add=True))
        for d in scats:
            d.wait()
        plsc.subcore_barrier()

        @pl.when(cid == 0)
        def _():
            @pl.when(par == 0)
            def _():
                pltpu.async_copy(_buf_slice(par),
                                 _out_slice(base_r, outw_hbm), sem_w0)
            @pl.when(par == 1)
            def _():
                pltpu.async_copy(_buf_slice(par),
                                 _out_slice(base_r, outw_hbm), sem_w1)
        @pl.when(cid == 1)
        def _():
            @pl.when(par == 0)
            def _():
                pltpu.async_copy(_buf_slice(par),
                                 _out_slice(base_r, outa_hbm), sem_w0)
            @pl.when(par == 1)
            def _():
                pltpu.async_copy(_buf_slice(par),
                                 _out_slice(base_r, outa_hbm), sem_w1)
        return ()

    lax.fori_loop(0, n_blocks, _block, ())

    _wait_writeback(jnp.int32(0), (n_blocks - 2) * blk_rows)
    _wait_writeback(jnp.int32(1), (n_blocks - 1) * blk_rows)


def _densify(rows, cols, w_vals, a_vals, n):
    nnz = rows.shape[0]
    blk_rows = 128
    blk_cells = blk_rows * n
    ept = nnz // _NS
    scat_w = 128

    mesh = plsc.VectorSubcoreMesh(core_axis_name="c", subcore_axis_name="s")
    body = functools.partial(_densify_body, n, nnz, blk_rows)
    out_w, out_a = pl.kernel(
        body,
        out_type=(jax.ShapeDtypeStruct((n * n,), jnp.float32),
                  jax.ShapeDtypeStruct((n * n,), jnp.float32)),
        mesh=mesh,
        scratch_types=[
            pltpu.VMEM((ept,), jnp.int32),
            pltpu.VMEM((ept,), jnp.int32),
            pltpu.VMEM((ept,), jnp.float32),
            pltpu.VMEM((ept,), jnp.float32),
            pltpu.VMEM((ept // scat_w, scat_w), jnp.int32),
            pltpu.VMEM((ept // scat_w, scat_w), jnp.float32),
            pltpu.VMEM((16384,), jnp.float32),
            pltpu.VMEM_SHARED((2 * blk_cells,), jnp.float32),
            pltpu.SemaphoreType.DMA,
            pltpu.SemaphoreType.DMA,
            pltpu.SemaphoreType.DMA,
            pltpu.SemaphoreType.DMA,
        ],
    )(rows, cols, w_vals, a_vals)
    return out_w.reshape(n, n), out_a.reshape(n, n)


def _mm_body(out_dtype, a_ref, b_ref, bias_ref, o_ref, acc_ref):
    @pl.when(pl.program_id(2) == 0)
    def _():
        acc_ref[...] = jnp.zeros_like(acc_ref)

    ah = a_ref[...].astype(jnp.bfloat16)
    bh = b_ref[...].astype(jnp.bfloat16)
    acc_ref[...] += jnp.dot(ah, bh, preferred_element_type=jnp.float32)

    @pl.when(pl.program_id(2) == pl.num_programs(2) - 1)
    def _():
        o_ref[...] = (acc_ref[...] + bias_ref[...]).astype(out_dtype)


def _mm_bias(a, b, bias2d, out_dtype, bm, bn, bk):
    n = a.shape[0]
    grid = (n // bm, n // bn, n // bk)
    return pl.pallas_call(
        functools.partial(_mm_body, out_dtype),
        grid=grid,
        in_specs=[
            pl.BlockSpec((bm, bk), lambda i, j, k: (i, k)),
            pl.BlockSpec((bk, bn), lambda i, j, k: (k, j)),
            pl.BlockSpec((1, bn), lambda i, j, k: (0, j)),
        ],
        out_specs=pl.BlockSpec((bm, bn), lambda i, j, k: (i, j)),
        out_shape=jax.ShapeDtypeStruct((n, n), out_dtype),
        scratch_shapes=[pltpu.VMEM((bm, bn), jnp.float32)],
        compiler_params=pltpu.CompilerParams(
            dimension_semantics=("parallel", "parallel", "arbitrary")),
    )(a, b, bias2d)


def kernel(x, rows, cols, a_vals, w_vals, bias):
    n = x.shape[0]
    w_d, a_d = _densify(rows, cols, w_vals, a_vals, n)
    x_bf = x.astype(jnp.bfloat16)
    zero_bias = jnp.zeros((1, n), jnp.float32)
    bias2d = bias.reshape(1, n)
    m = _mm_bias(a_d, w_d, zero_bias, jnp.bfloat16, 2048, 1024, 1024)
    out = _mm_bias(m, x_bf, bias2d, jnp.bfloat16, 2048, 2048, 512)
    out = _mm_bias(m, out, bias2d, jnp.float32, 2048, 1024, 512)
    return out


# speedup vs baseline: 19.9812x; 1.0254x over previous
import functools

import jax
import jax.numpy as jnp
from jax import lax
from jax.experimental import pallas as pl
from jax.experimental.pallas import tpu as pltpu
from jax.experimental.pallas import tpu_sc as plsc

_NC = 2
_NS = 16
_LANES = 16


def _densify_body(n, nnz, blk_rows, rows_hbm, cols_hbm, wvals_hbm, avals_hbm,
                  outw_hbm, outa_hbm,
                  rows_v, cols_v, wv_v, av_v, offs_v, upds_v, zbuf_v, blk_sh,
                  sem_c, sem_s, sem_w0, sem_w1):
    cid = lax.axis_index("c")
    sid = lax.axis_index("s")
    ept = nnz // _NS
    blk_cells = blk_rows * n
    cells_per_tec = blk_cells // _NS
    n_blocks = n // blk_rows
    n_vregs = ept // _LANES
    scat_w = offs_v.shape[1]
    vregs_per_scat = scat_w // _LANES

    base_e = sid * ept
    pltpu.sync_copy(rows_hbm.at[pl.ds(base_e, ept)], rows_v)
    pltpu.sync_copy(cols_hbm.at[pl.ds(base_e, ept)], cols_v)
    pltpu.sync_copy(wvals_hbm.at[pl.ds(base_e, ept)], wv_v)
    pltpu.sync_copy(avals_hbm.at[pl.ds(base_e, ept)], av_v)

    zchunk = zbuf_v.shape[0]
    def _z(i, _):
        zbuf_v[pl.ds(i * _LANES, _LANES)] = jnp.zeros((_LANES,), jnp.float32)
        return ()
    lax.fori_loop(0, zchunk // _LANES, _z, ())

    rows_per_tec = blk_rows // _NS

    def _out_slice(base_r, out_hbm):
        return _out_flat_slice(base_r, out_hbm)

    def _out_flat_slice(base_r, out_hbm):
        return out_hbm.at[pl.ds(base_r * n + sid * cells_per_tec,
                                cells_per_tec)]

    def _buf_slice(par):
        return blk_sh.at[pl.ds(par * blk_cells + sid * cells_per_tec,
                               cells_per_tec)]

    def _wait_writeback(par, base_r_prev):
        @pl.when(cid == 0)
        def _():
            @pl.when(par == 0)
            def _():
                pltpu.make_async_copy(
                    _buf_slice(par), _out_slice(base_r_prev, outw_hbm),
                    sem_w0).wait()
            @pl.when(par == 1)
            def _():
                pltpu.make_async_copy(
                    _buf_slice(par), _out_slice(base_r_prev, outw_hbm),
                    sem_w1).wait()
        @pl.when(cid == 1)
        def _():
            @pl.when(par == 0)
            def _():
                pltpu.make_async_copy(
                    _buf_slice(par), _out_slice(base_r_prev, outa_hbm),
                    sem_w0).wait()
            @pl.when(par == 1)
            def _():
                pltpu.make_async_copy(
                    _buf_slice(par), _out_slice(base_r_prev, outa_hbm),
                    sem_w1).wait()

    def _block(bi, _):
        base_r = bi * blk_rows
        par = lax.rem(bi, 2)
        buf_base = par * blk_cells

        @pl.when(bi >= 2)
        def _():
            _wait_writeback(par, base_r - 2 * blk_rows)

        clrs = []
        for i in range(cells_per_tec // zchunk):
            clrs.append(pltpu.async_copy(
                zbuf_v,
                blk_sh.at[pl.ds(buf_base + sid * cells_per_tec + i * zchunk,
                                zchunk)],
                sem_c))

        def _prep(v, _):
            r = rows_v[pl.ds(v * _LANES, _LANES)]
            c = cols_v[pl.ds(v * _LANES, _LANES)]
            wv = wv_v[pl.ds(v * _LANES, _LANES)]
            av = av_v[pl.ds(v * _LANES, _LANES)]
            w = jnp.where(cid == 0, wv, av)
            rel = r - base_r
            inblk = (rel >= 0) & (rel < blk_rows)
            dummy = sid * ept + v * _LANES + lax.iota(jnp.int32, _LANES)
            off = buf_base + jnp.where(inblk, rel * n + c, dummy)
            val = jnp.where(inblk, w, jnp.zeros((_LANES,), jnp.float32))
            row = v // vregs_per_scat
            colo = (v % vregs_per_scat) * _LANES
            offs_v[row, pl.ds(colo, _LANES)] = off
            upds_v[row, pl.ds(colo, _LANES)] = val
            return ()
        lax.fori_loop(0, n_vregs, _prep, ())
        for d in clrs:
            d.wait()
        plsc.subcore_barrier()

        scats = []
        for j in range(ept // scat_w):
            scats.append(pltpu.async_copy(
                upds_v.at[j], blk_sh.at[offs_v.at[j]], sem_s, add=True))
        for d in scats:
            d.wait()
        plsc.subcore_barrier()

        @pl.when(cid == 0)
        def _():
            @pl.when(par == 0)
            def _():
                pltpu.async_copy(_buf_slice(par),
                                 _out_slice(base_r, outw_hbm), sem_w0)
            @pl.when(par == 1)
            def _():
                pltpu.async_copy(_buf_slice(par),
                                 _out_slice(base_r, outw_hbm), sem_w1)
        @pl.when(cid == 1)
        def _():
            @pl.when(par == 0)
            def _():
                pltpu.async_copy(_buf_slice(par),
                                 _out_slice(base_r, outa_hbm), sem_w0)
            @pl.when(par == 1)
            def _():
                pltpu.async_copy(_buf_slice(par),
                                 _out_slice(base_r, outa_hbm), sem_w1)
        return ()

    lax.fori_loop(0, n_blocks, _block, ())

    _wait_writeback(jnp.int32(0), (n_blocks - 2) * blk_rows)
    _wait_writeback(jnp.int32(1), (n_blocks - 1) * blk_rows)


def _densify(rows, cols, w_vals, a_vals, n):
    nnz = rows.shape[0]
    blk_rows = 128
    blk_cells = blk_rows * n
    ept = nnz // _NS
    scat_w = 128

    mesh = plsc.VectorSubcoreMesh(core_axis_name="c", subcore_axis_name="s")
    body = functools.partial(_densify_body, n, nnz, blk_rows)
    out_w, out_a = pl.kernel(
        body,
        out_type=(jax.ShapeDtypeStruct((n * n,), jnp.float32),
                  jax.ShapeDtypeStruct((n * n,), jnp.float32)),
        mesh=mesh,
        scratch_types=[
            pltpu.VMEM((ept,), jnp.int32),
            pltpu.VMEM((ept,), jnp.int32),
            pltpu.VMEM((ept,), jnp.float32),
            pltpu.VMEM((ept,), jnp.float32),
            pltpu.VMEM((ept // scat_w, scat_w), jnp.int32),
            pltpu.VMEM((ept // scat_w, scat_w), jnp.float32),
            pltpu.VMEM((16384,), jnp.float32),
            pltpu.VMEM_SHARED((2 * blk_cells,), jnp.float32),
            pltpu.SemaphoreType.DMA,
            pltpu.SemaphoreType.DMA,
            pltpu.SemaphoreType.DMA,
            pltpu.SemaphoreType.DMA,
        ],
    )(rows, cols, w_vals, a_vals)
    return out_w.reshape(n, n), out_a.reshape(n, n)


def _mm_body(out_dtype, a_ref, b_ref, bias_ref, o_ref, acc_ref):
    @pl.when(pl.program_id(2) == 0)
    def _():
        acc_ref[...] = jnp.zeros_like(acc_ref)

    ah = a_ref[...].astype(jnp.bfloat16)
    bh = b_ref[...].astype(jnp.bfloat16)
    acc_ref[...] += jnp.dot(ah, bh, preferred_element_type=jnp.float32)

    @pl.when(pl.program_id(2) == pl.num_programs(2) - 1)
    def _():
        o_ref[...] = (acc_ref[...] + bias_ref[...]).astype(out_dtype)


def _mm_bias(a, b, bias2d, out_dtype, bm, bn, bk):
    n = a.shape[0]
    grid = (n // bm, n // bn, n // bk)
    return pl.pallas_call(
        functools.partial(_mm_body, out_dtype),
        grid=grid,
        in_specs=[
            pl.BlockSpec((bm, bk), lambda i, j, k: (i, k)),
            pl.BlockSpec((bk, bn), lambda i, j, k: (k, j)),
            pl.BlockSpec((1, bn), lambda i, j, k: (0, j)),
        ],
        out_specs=pl.BlockSpec((bm, bn), lambda i, j, k: (i, j)),
        out_shape=jax.ShapeDtypeStruct((n, n), out_dtype),
        scratch_shapes=[pltpu.VMEM((bm, bn), jnp.float32)],
        compiler_params=pltpu.CompilerParams(
            dimension_semantics=("parallel", "parallel", "arbitrary")),
    )(a, b, bias2d)


def kernel(x, rows, cols, a_vals, w_vals, bias):
    n = x.shape[0]
    w_d, a_d = _densify(rows, cols, w_vals, a_vals, n)
    x_bf = x.astype(jnp.bfloat16)
    zero_bias = jnp.zeros((1, n), jnp.float32)
    bias2d = bias.reshape(1, n)
    m = _mm_bias(a_d, w_d, zero_bias, jnp.bfloat16, 2048, 1024, 1024)
    out = _mm_bias(m, x_bf, bias2d, jnp.bfloat16, 2048, 2048, 1024)
    out = _mm_bias(m, out, bias2d, jnp.float32, 1024, 2048, 1024)
    return out
